# baseline (speedup 1.0000x reference)
import functools

import jax
import jax.numpy as jnp
from jax import lax
from jax.experimental import pallas as pl
from jax.experimental.pallas import tpu as pltpu
from jax.experimental.pallas import tpu_sc as plsc

D = 128
IDX_CHUNK = 128


def _sc_gather(table, idx3, n_workers, n_chunks):
    b_per_w = n_chunks * IDX_CHUNK
    B = n_workers * b_per_w
    mesh = plsc.VectorSubcoreMesh(core_axis_name="c", subcore_axis_name="s")

    @functools.partial(
        pl.kernel,
        mesh=mesh,
        out_type=jax.ShapeDtypeStruct((B, D), jnp.float32),
        scratch_types=[
            pltpu.VMEM((n_chunks, IDX_CHUNK), jnp.int32),
            pltpu.VMEM((b_per_w, D), jnp.float32),
            pltpu.SemaphoreType.DMA,
        ],
    )
    def gather_kernel(table_hbm, idx_hbm, out_hbm, idx_v, rows_v, sem):
        wid = lax.axis_index("s") * 2 + lax.axis_index("c")
        pltpu.sync_copy(idx_hbm.at[wid], idx_v)
        copies = []
        for j in range(n_chunks):
            copies.append(
                pltpu.async_copy(
                    table_hbm.at[idx_v.at[j]],
                    rows_v.at[pl.ds(j * IDX_CHUNK, IDX_CHUNK)],
                    sem,
                )
            )
        for c in copies:
            c.wait()
        pltpu.sync_copy(rows_v, out_hbm.at[pl.ds(wid * b_per_w, b_per_w)])

    return gather_kernel(table, idx3)


def _mlp_block(dev_emb_ref, clsf_ref, ct_ref, w1_ref, b1_ref,
               w2_ref, b2_ref, out_ref):
    cmat = jnp.dot(ct_ref[...], w1_ref[:D],
                   preferred_element_type=jnp.float32,
                   precision=lax.Precision.DEFAULT)
    clsi = clsf_ref[...].astype(jnp.int32)
    c_sel = jnp.where(
        clsi == 1, cmat[1:2, :],
        jnp.where(clsi == 2, cmat[2:3, :], cmat[0:1, :]))
    h = jnp.dot(dev_emb_ref[...], w1_ref[D:],
                preferred_element_type=jnp.float32,
                precision=lax.Precision.DEFAULT)
    h = h + c_sel + b1_ref[...]
    h = h * jax.nn.sigmoid(h)
    out = jnp.dot(h, w2_ref[...],
                  preferred_element_type=jnp.float32,
                  precision=lax.Precision.DEFAULT)
    out_ref[...] = out + b2_ref[...]


def _mlp(dev_emb, clsf, ct, w1, b1, w2, b2, blk):
    B = dev_emb.shape[0]
    grid = B // blk
    return pl.pallas_call(
        _mlp_block,
        grid=(grid,),
        in_specs=[
            pl.BlockSpec((blk, D), lambda i: (i, 0)),
            pl.BlockSpec((blk, 1), lambda i: (i, 0)),
            pl.BlockSpec((3, D), lambda i: (0, 0)),
            pl.BlockSpec((2 * D, D), lambda i: (0, 0)),
            pl.BlockSpec((1, D), lambda i: (0, 0)),
            pl.BlockSpec((D, D), lambda i: (0, 0)),
            pl.BlockSpec((1, D), lambda i: (0, 0)),
        ],
        out_specs=pl.BlockSpec((blk, D), lambda i: (i, 0)),
        out_shape=jax.ShapeDtypeStruct((B, D), jnp.float32),
    )(dev_emb, clsf, ct, w1, b1, w2, b2)


def kernel(box, cls, dev, cls_table, dev_table, W1, b1, W2, b2):
    B = dev.shape[0]
    n_workers = 32
    n_chunks = B // (n_workers * IDX_CHUNK)
    idx3 = dev.astype(jnp.int32).reshape(n_workers, n_chunks, IDX_CHUNK)
    dev_emb = _sc_gather(dev_table, idx3, n_workers, n_chunks)

    cls8 = cls.astype(jnp.int8).reshape(B, 1)
    return _mlp(dev_emb, cls8, cls_table, W1,
                b1.reshape(1, D), W2, b2.reshape(1, D), blk=2048)


# speedup vs baseline: 1.0557x; 1.0557x over previous
import functools

import jax
import jax.numpy as jnp
from jax import lax
from jax.experimental import pallas as pl
from jax.experimental.pallas import tpu as pltpu
from jax.experimental.pallas import tpu_sc as plsc

D = 128
IDX_CHUNK = 128


def _sc_gather(table, idx3, n_workers, n_chunks):
    b_per_w = n_chunks * IDX_CHUNK
    B = n_workers * b_per_w
    mesh = plsc.VectorSubcoreMesh(core_axis_name="c", subcore_axis_name="s")

    @functools.partial(
        pl.kernel,
        mesh=mesh,
        out_type=jax.ShapeDtypeStruct((B, D), jnp.float32),
        scratch_types=[
            pltpu.VMEM((n_chunks, IDX_CHUNK), jnp.int32),
            pltpu.VMEM((b_per_w, D), jnp.float32),
            pltpu.SemaphoreType.DMA,
        ],
    )
    def gather_kernel(table_hbm, idx_hbm, out_hbm, idx_v, rows_v, sem):
        wid = lax.axis_index("s") * 2 + lax.axis_index("c")
        pltpu.sync_copy(idx_hbm.at[wid], idx_v)
        copies = []
        for j in range(n_chunks):
            copies.append(
                pltpu.async_copy(
                    table_hbm.at[idx_v.at[j]],
                    rows_v.at[pl.ds(j * IDX_CHUNK, IDX_CHUNK)],
                    sem,
                )
            )
        for c in copies:
            c.wait()
        pltpu.sync_copy(rows_v, out_hbm.at[pl.ds(wid * b_per_w, b_per_w)])

    return gather_kernel(table, idx3)


def _mlp_block(dev_emb_ref, clsf_ref, ct_ref, w1_ref, b1_ref,
               w2_ref, b2_ref, out_ref):
    cmat = jnp.dot(ct_ref[...], w1_ref[:D],
                   preferred_element_type=jnp.float32,
                   precision=lax.Precision.DEFAULT)
    clsi = clsf_ref[...].astype(jnp.int32)
    c_sel = jnp.where(
        clsi == 1, cmat[1:2, :],
        jnp.where(clsi == 2, cmat[2:3, :], cmat[0:1, :]))
    h = jnp.dot(dev_emb_ref[...], w1_ref[D:],
                preferred_element_type=jnp.float32,
                precision=lax.Precision.DEFAULT)
    h = h + c_sel + b1_ref[...]
    h = h * jax.nn.sigmoid(h)
    out = jnp.dot(h, w2_ref[...],
                  preferred_element_type=jnp.float32,
                  precision=lax.Precision.DEFAULT)
    out_ref[...] = out + b2_ref[...]


def _mlp(dev_emb, clsf, ct, w1, b1, w2, b2, blk):
    B = dev_emb.shape[0]
    grid = B // blk
    return pl.pallas_call(
        _mlp_block,
        grid=(grid,),
        in_specs=[
            pl.BlockSpec((blk, D), lambda i: (i, 0)),
            pl.BlockSpec((blk, 1), lambda i: (i, 0)),
            pl.BlockSpec((3, D), lambda i: (0, 0)),
            pl.BlockSpec((2 * D, D), lambda i: (0, 0)),
            pl.BlockSpec((1, D), lambda i: (0, 0)),
            pl.BlockSpec((D, D), lambda i: (0, 0)),
            pl.BlockSpec((1, D), lambda i: (0, 0)),
        ],
        out_specs=pl.BlockSpec((blk, D), lambda i: (i, 0)),
        out_shape=jax.ShapeDtypeStruct((B, D), jnp.float32),
    )(dev_emb, clsf, ct, w1, b1, w2, b2)


def kernel(box, cls, dev, cls_table, dev_table, W1, b1, W2, b2):
    B = dev.shape[0]
    n_workers = 32
    n_chunks = B // (n_workers * IDX_CHUNK)
    idx3 = dev.astype(jnp.int32).reshape(n_workers, n_chunks, IDX_CHUNK)
    dev_emb = _sc_gather(dev_table, idx3, n_workers, n_chunks)

    cls8 = cls.astype(jnp.int8).reshape(B, 1)
    return _mlp(dev_emb, cls8, cls_table, W1,
                b1.reshape(1, D), W2, b2.reshape(1, D), blk=4096)
